# Initial kernel scaffold; baseline (speedup 1.0000x reference)
import jax
import jax.numpy as jnp
from jax.experimental import pallas as pl


def kernel(x, node_feature_extra, edge_index, edge_type, node_type, W_enc1, b_enc1, W_enc2, b_enc2, W_key, b_key, W_msg, b_msg, W_query, b_query, W_mlp1, b_mlp1, bn_gamma, bn_beta, W_mlp2, b_mlp2):
    raise NotImplementedError("write your pallas kernel here")


# speedup vs baseline: 2.3646x; 2.3646x over previous
import functools
import math

import jax
import jax.numpy as jnp
from jax import lax
from jax.experimental import pallas as pl
from jax.experimental.pallas import tpu as pltpu
from jax.experimental.pallas import tpu_sc as plsc

N = 10000
EMB = 128
NT = 4
ET = 38
H = 4
DH = EMB // H
TAB = (ET + 1) * NT * NT
NPAD = 10240
C = 96
CPW = 108
NW = 32
EPW = C * CPW
EPAD = NW * EPW
RPT = NPAD // 16

_f32 = jnp.float32
_i32 = jnp.int32


def _dense_body(xx, enc, we1, be1, we2, be2, wkx, wke, bk, wmx, wme, bm,
                wq, bq, qn_o, kn_o, mn_o, kc_o, mc_o):
    eh = jnp.maximum(
        jnp.dot(enc[...], we1[...], preferred_element_type=_f32) + be1[...], 0.0)
    eemb = jnp.dot(eh, we2[...], preferred_element_type=_f32) + be2[...]
    kc_o[...] = jnp.dot(eemb, wke[...], preferred_element_type=_f32) + bk[...]
    mc_o[...] = jnp.dot(eemb, wme[...], preferred_element_type=_f32) + bm[...]
    xv = xx[...]
    qn_o[...] = (jnp.dot(xv, wq[...], preferred_element_type=_f32)
                 + bq[...]) * (1.0 / math.sqrt(DH))
    kn_o[...] = jnp.dot(xv, wkx[...], preferred_element_type=_f32)
    mn_o[...] = jnp.dot(xv, wmx[...], preferred_element_type=_f32)


_dense_call = pl.pallas_call(
    _dense_body,
    out_shape=[
        jax.ShapeDtypeStruct((N, EMB), _f32),
        jax.ShapeDtypeStruct((N, EMB), _f32),
        jax.ShapeDtypeStruct((N, EMB), _f32),
        jax.ShapeDtypeStruct((TAB, EMB), _f32),
        jax.ShapeDtypeStruct((TAB, EMB), _f32),
    ],
)


def _mlp_body(aggr, w1, b1, gamma, beta, w2, b2, out):
    a = aggr[0:N, :] + aggr[NPAD:NPAD + N, :]
    h1 = jnp.dot(a, w1[...], preferred_element_type=_f32) + b1[...]
    mu = jnp.mean(h1, axis=0, keepdims=True)
    d = h1 - mu
    var = jnp.mean(d * d, axis=0, keepdims=True)
    h1 = d * lax.rsqrt(var + 1e-5) * gamma[...] + beta[...]
    h1 = jnp.maximum(h1, 0.0)
    out[...] = jnp.dot(h1, w2[...], preferred_element_type=_f32) + b2[...]


_mlp_call = pl.pallas_call(
    _mlp_body,
    out_shape=jax.ShapeDtypeStruct((N, EMB), _f32),
)


def _pass1_body(qn, kn, kc, src, dst, eta, ntp, zsums,
                ex_out, sums_out, combo_out,
                kc_l, nt_l, qbuf, kbuf, csums, sidx, didx, etb, cbuf,
                sums_sh, sem1, sem2):
    cid = lax.axis_index("c")
    sid = lax.axis_index("s")
    wid = sid * 2 + cid
    lanes = jnp.arange(16, dtype=_i32)
    ones16 = jnp.ones((16,), _f32)
    zero16 = jnp.zeros((16,), _f32)

    pltpu.sync_copy(kc, kc_l)
    pltpu.sync_copy(ntp, nt_l)
    row0 = sid * RPT
    pltpu.sync_copy(zsums.at[pl.ds(row0, RPT)], sums_sh.at[pl.ds(row0, RPT)])
    for r in range(C):
        csums[r, :] = zero16
    plsc.subcore_barrier()

    def chunk_body(ci, _):
        base = wid * EPW + ci * C
        pltpu.sync_copy(src.at[pl.ds(base, C)], sidx)
        pltpu.sync_copy(dst.at[pl.ds(base, C)], didx)
        pltpu.sync_copy(eta.at[pl.ds(base, C)], etb)
        cp1 = pltpu.async_copy(qn.at[sidx], qbuf, sem1)
        cp2 = pltpu.async_copy(kn.at[didx], kbuf, sem2)
        cp1.wait()
        cp2.wait()
        for g in range(C // 16):
            rows = lanes + g * 16
            s16 = sidx[pl.ds(g * 16, 16)]
            d16 = didx[pl.ds(g * 16, 16)]
            e16 = etb[pl.ds(g * 16, 16)]
            nts = plsc.load_gather(nt_l, [s16 >> 4, s16 & 15])
            ntd = plsc.load_gather(nt_l, [d16 >> 4, d16 & 15])
            combo = e16 * (NT * NT) + nts * NT + ntd
            cbuf[pl.ds(g * 16, 16)] = combo

            def dot_body(j, accs):
                res = []
                for h in range(H):
                    dcol = jnp.full((16,), h * DH, _i32) + j
                    q = plsc.load_gather(qbuf, [rows, dcol])
                    k = plsc.load_gather(kbuf, [rows, dcol])
                    kcv = plsc.load_gather(kc_l, [combo, dcol])
                    res.append(accs[h] + q * (k + kcv))
                return tuple(res)

            accs = lax.fori_loop(0, DH, dot_body,
                                 (zero16, zero16, zero16, zero16))
            for h in range(H):
                exh = jnp.exp(accs[h])
                plsc.store_scatter(csums, [rows, jnp.full((16,), h, _i32)], exh)
            plsc.store_scatter(csums, [rows, jnp.full((16,), H, _i32)], ones16)
        pltpu.sync_copy(csums, ex_out.at[pl.ds(base, C)])
        pltpu.sync_copy(cbuf, combo_out.at[pl.ds(base, C)])
        pltpu.sync_copy(csums, sums_sh.at[sidx], add=True)
        return 0

    lax.fori_loop(0, CPW, chunk_body, 0)
    plsc.subcore_barrier()
    pltpu.sync_copy(sums_sh.at[pl.ds(row0, RPT)],
                    sums_out.at[pl.ds(cid * NPAD + row0, RPT)])


_pass1 = pl.kernel(
    _pass1_body,
    out_type=[
        jax.ShapeDtypeStruct((EPAD, 16), _f32),
        jax.ShapeDtypeStruct((2 * NPAD, 16), _f32),
        jax.ShapeDtypeStruct((EPAD,), _i32),
    ],
    mesh=plsc.VectorSubcoreMesh(core_axis_name="c", subcore_axis_name="s",
                                num_cores=2, num_subcores=16),
    scratch_types=[
        pltpu.VMEM((TAB, EMB), _f32),
        pltpu.VMEM((NPAD // 16, 16), _i32),
        pltpu.VMEM((C, EMB), _f32),
        pltpu.VMEM((C, EMB), _f32),
        pltpu.VMEM((C, 16), _f32),
        pltpu.VMEM((C,), _i32),
        pltpu.VMEM((C,), _i32),
        pltpu.VMEM((C,), _i32),
        pltpu.VMEM((C,), _i32),
        pltpu.MemorySpace.VMEM_SHARED((NPAD, 16), _f32),
        pltpu.SemaphoreType.DMA,
        pltpu.SemaphoreType.DMA,
    ],
    compiler_params=pltpu.CompilerParams(needs_layout_passes=False,
                                         use_tc_tiling_on_sc=False),
)


def _pass2_body(mn, mc, src, dst, combo_in, ex_in, sums, zaggr,
                aggr_out,
                mbuf, mcbuf, outch, s0b, s1b, s1idx, exch, sidx, didx,
                cbuf, aggr_sh, sem1, sem2):
    cid = lax.axis_index("c")
    sid = lax.axis_index("s")
    wid = sid * 2 + cid
    lanes = jnp.arange(16, dtype=_i32)

    row0 = sid * RPT
    pltpu.sync_copy(zaggr.at[pl.ds(row0, RPT)], aggr_sh.at[pl.ds(row0, RPT)])
    plsc.subcore_barrier()

    def chunk_body(ci, _):
        base = wid * EPW + ci * C
        pltpu.sync_copy(src.at[pl.ds(base, C)], sidx)
        pltpu.sync_copy(dst.at[pl.ds(base, C)], didx)
        pltpu.sync_copy(combo_in.at[pl.ds(base, C)], cbuf)
        pltpu.sync_copy(ex_in.at[pl.ds(base, C)], exch)
        for t in range(C // 16):
            s1idx[pl.ds(t * 16, 16)] = sidx[pl.ds(t * 16, 16)] + NPAD
        cp1 = pltpu.async_copy(mn.at[sidx], mbuf, sem1)
        cp2 = pltpu.async_copy(sums.at[sidx], s0b, sem2)
        cp1.wait()
        cp2.wait()
        cp3 = pltpu.async_copy(mc.at[cbuf], mcbuf, sem1)
        cp4 = pltpu.async_copy(sums.at[s1idx], s1b, sem2)
        cp3.wait()
        cp4.wait()
        for g in range(C // 16):
            rows = lanes + g * 16
            colH = jnp.full((16,), H, _i32)
            cnt = (plsc.load_gather(s0b, [rows, colH])
                   + plsc.load_gather(s1b, [rows, colH]))
            alphas = []
            for h in range(H):
                colh = jnp.full((16,), h, _i32)
                ssum = (plsc.load_gather(s0b, [rows, colh])
                        + plsc.load_gather(s1b, [rows, colh]))
                exh = plsc.load_gather(exch, [rows, colh])
                alphas.append(exh * cnt / (ssum + 1e-16))

            def col_body(j, _):
                for h in range(H):
                    dcol = jnp.full((16,), h * DH, _i32) + j
                    m = (plsc.load_gather(mbuf, [rows, dcol])
                         + plsc.load_gather(mcbuf, [rows, dcol]))
                    plsc.store_scatter(outch, [rows, dcol], m * alphas[h])
                return 0

            lax.fori_loop(0, DH, col_body, 0)
        pltpu.sync_copy(outch, aggr_sh.at[didx], add=True)
        return 0

    lax.fori_loop(0, CPW, chunk_body, 0)
    plsc.subcore_barrier()
    pltpu.sync_copy(aggr_sh.at[pl.ds(row0, RPT)],
                    aggr_out.at[pl.ds(cid * NPAD + row0, RPT)])


_pass2 = pl.kernel(
    _pass2_body,
    out_type=jax.ShapeDtypeStruct((2 * NPAD, EMB), _f32),
    mesh=plsc.VectorSubcoreMesh(core_axis_name="c", subcore_axis_name="s",
                                num_cores=2, num_subcores=16),
    scratch_types=[
        pltpu.VMEM((C, EMB), _f32),
        pltpu.VMEM((C, EMB), _f32),
        pltpu.VMEM((C, EMB), _f32),
        pltpu.VMEM((C, 16), _f32),
        pltpu.VMEM((C, 16), _f32),
        pltpu.VMEM((C,), _i32),
        pltpu.VMEM((C, 16), _f32),
        pltpu.VMEM((C,), _i32),
        pltpu.VMEM((C,), _i32),
        pltpu.VMEM((C,), _i32),
        pltpu.MemorySpace.VMEM_SHARED((NPAD, EMB), _f32),
        pltpu.SemaphoreType.DMA,
        pltpu.SemaphoreType.DMA,
    ],
    compiler_params=pltpu.CompilerParams(needs_layout_passes=False,
                                         use_tc_tiling_on_sc=False),
)


def kernel(x, node_feature_extra, edge_index, edge_type, node_type,
           W_enc1, b_enc1, W_enc2, b_enc2,
           W_key, b_key, W_msg, b_msg, W_query, b_query,
           W_mlp1, b_mlp1, bn_gamma, bn_beta, W_mlp2, b_mlp2):
    E = edge_index.shape[1]
    xx = jnp.concatenate([x, node_feature_extra], axis=1)

    cidx = jnp.arange(TAB)
    et = cidx // (NT * NT)
    ht = (cidx // NT) % NT
    tt = cidx % NT
    enc = jnp.concatenate([
        jax.nn.one_hot(et, ET + 1, dtype=_f32),
        jax.nn.one_hot(ht, NT, dtype=_f32),
        jax.nn.one_hot(tt, NT, dtype=_f32),
    ], axis=1)
    enc = jnp.pad(enc, ((0, 0), (0, 1)))
    we1 = jnp.pad(W_enc1, ((0, 1), (0, 0)))

    qn, kn, mn, kc, mc = _dense_call(
        xx, enc, we1, b_enc1.reshape(1, -1), W_enc2, b_enc2.reshape(1, -1),
        W_key[:2 * EMB], W_key[2 * EMB:], b_key.reshape(1, -1),
        W_msg[:2 * EMB], W_msg[2 * EMB:], b_msg.reshape(1, -1),
        W_query, b_query.reshape(1, -1))

    qn_p = jnp.pad(qn, ((0, NPAD - N), (0, 0)))
    kn_p = jnp.pad(kn, ((0, NPAD - N), (0, 0)))
    mn_p = jnp.pad(mn, ((0, NPAD - N), (0, 0)))

    loop = jnp.arange(N, dtype=_i32)
    pad_e = EPAD - (E + N)
    src = jnp.concatenate([edge_index[0], loop, jnp.full((pad_e,), N, _i32)])
    dst = jnp.concatenate([edge_index[1], loop, jnp.full((pad_e,), N, _i32)])
    eta = jnp.concatenate([edge_type.astype(_i32), jnp.full((N,), ET, _i32),
                           jnp.full((pad_e,), ET, _i32)])
    ntp = jnp.concatenate([node_type.astype(_i32),
                           jnp.zeros((NPAD - N,), _i32)]).reshape(NPAD // 16, 16)
    zsums = jnp.zeros((NPAD, 16), _f32)
    zaggr = jnp.zeros((NPAD, EMB), _f32)

    ex_flat, sums, combo = _pass1(qn_p, kn_p, kc, src, dst, eta, ntp, zsums)
    aggr2 = _pass2(mn_p, mc, src, dst, combo, ex_flat, sums, zaggr)

    return _mlp_call(aggr2, W_mlp1, b_mlp1.reshape(1, -1),
                     bn_gamma.reshape(1, -1), bn_beta.reshape(1, -1),
                     W_mlp2, b_mlp2.reshape(1, -1))
